# Initial kernel scaffold; baseline (speedup 1.0000x reference)
import jax
import jax.numpy as jnp
from jax.experimental import pallas as pl


def kernel(x, router_w, w1, w2):
    raise NotImplementedError("write your pallas kernel here")


# speedup vs baseline: 1.1779x; 1.1779x over previous
import functools

import jax
import jax.numpy as jnp
from jax import lax
from jax.experimental import pallas as pl
from jax.experimental.pallas import tpu as pltpu

T = 2048
D = 1024
E = 8
H = 4096
K = 2
KH = 4
HC = H // KH


def _router_body(x_ref, rw_ref, comb_ref):
    x = x_ref[...]
    rw = rw_ref[...]
    logits = jnp.dot(x, rw, preferred_element_type=jnp.float32)
    m = jnp.max(logits, axis=-1, keepdims=True)
    ex = jnp.exp(logits - m)
    scores = ex / jnp.sum(ex, axis=-1, keepdims=True)

    eids = lax.broadcasted_iota(jnp.int32, (T, E), 1)
    m1 = jnp.max(scores, axis=-1, keepdims=True)
    i1 = jnp.argmax(scores, axis=-1)[:, None]
    masked = jnp.where(eids == i1, -jnp.inf, scores)
    m2 = jnp.max(masked, axis=-1, keepdims=True)
    i2 = jnp.argmax(masked, axis=-1)[:, None]

    comb_ref[...] = jnp.where(eids == i1, m1, 0.0) + jnp.where(eids == i2, m2, 0.0)


def _expert_body(x_ref, w1_ref, w2_ref, comb_ref, out_ref):
    e = pl.program_id(0)
    kh = pl.program_id(1)

    eids = lax.broadcasted_iota(jnp.int32, (T, E), 1)
    c = jnp.sum(comb_ref[...] * (eids == e).astype(jnp.float32),
                axis=-1, keepdims=True)

    h = jnp.dot(x_ref[...], w1_ref[0], preferred_element_type=jnp.float32)
    h = h * jax.nn.sigmoid(h)
    contrib = jnp.dot(h * c, w2_ref[0], preferred_element_type=jnp.float32)

    @pl.when(jnp.logical_and(e == 0, kh == 0))
    def _():
        out_ref[...] = contrib

    @pl.when(jnp.logical_not(jnp.logical_and(e == 0, kh == 0)))
    def _():
        out_ref[...] += contrib


@jax.jit
def kernel(x, router_w, w1, w2):
    combine = pl.pallas_call(
        _router_body,
        out_shape=jax.ShapeDtypeStruct((T, E), jnp.float32),
    )(x, router_w)

    out = pl.pallas_call(
        _expert_body,
        grid=(E, KH),
        in_specs=[
            pl.BlockSpec((T, D), lambda e, kh: (0, 0)),
            pl.BlockSpec((1, D, HC), lambda e, kh: (e, 0, kh)),
            pl.BlockSpec((1, HC, D), lambda e, kh: (e, kh, 0)),
            pl.BlockSpec((T, E), lambda e, kh: (0, 0)),
        ],
        out_specs=pl.BlockSpec((T, D), lambda e, kh: (0, 0)),
        out_shape=jax.ShapeDtypeStruct((T, D), jnp.float32),
    )(x, w1, w2, combine)
    return out


# speedup vs baseline: 1.2011x; 1.0197x over previous
import functools

import jax
import jax.numpy as jnp
from jax import lax
from jax.experimental import pallas as pl
from jax.experimental.pallas import tpu as pltpu
from jax.experimental.pallas import tpu_sc as plsc

T = 2048
D = 1024
E = 8
H = 4096
K = 2
KH = 4
HC = H // KH
BT = 128
NT = 40
PAD = NT * BT

NSUB = 32
TPW = T // NSUB


def _router_body(x_ref, rw_ref, s0_ref, s1_ref, w0_ref, w1_ref, te_ref):
    x = x_ref[...]
    logits = jnp.dot(x, rw_ref[...], preferred_element_type=jnp.float32)
    m = jnp.max(logits, axis=-1, keepdims=True)
    ex = jnp.exp(logits - m)
    scores = ex / jnp.sum(ex, axis=-1, keepdims=True)

    eids = lax.broadcasted_iota(jnp.int32, (T, E), 1)
    m1 = jnp.max(scores, axis=-1, keepdims=True)
    i1 = jnp.argmax(scores, axis=-1)[:, None]
    masked = jnp.where(eids == i1, -jnp.inf, scores)
    m2 = jnp.max(masked, axis=-1, keepdims=True)
    i2 = jnp.argmax(masked, axis=-1)[:, None]

    sel1 = (eids == i1)
    sel2 = (eids == i2)
    mask = (sel1 | sel2).astype(jnp.float32)

    r = lax.broadcasted_iota(jnp.int32, (T, T), 0)
    c = lax.broadcasted_iota(jnp.int32, (T, T), 1)
    ltri = (c < r).astype(jnp.bfloat16)
    excl = jnp.dot(ltri, mask.astype(jnp.bfloat16),
                   preferred_element_type=jnp.float32)

    counts = jnp.sum(mask, axis=0, keepdims=True)
    cnt_pad = (jnp.ceil(counts / BT) * BT).astype(jnp.float32)

    er = lax.broadcasted_iota(jnp.int32, (E, E), 0)
    ec = lax.broadcasted_iota(jnp.int32, (E, E), 1)
    off = jnp.dot(cnt_pad, (er < ec).astype(jnp.float32),
                  preferred_element_type=jnp.float32)

    slot_base = off + excl
    slot1 = jnp.sum(jnp.where(sel1, slot_base, 0.0), axis=-1, keepdims=True)
    slot2 = jnp.sum(jnp.where(sel2, slot_base, 0.0), axis=-1, keepdims=True)
    s0_ref[...] = slot1.astype(jnp.int32)
    s1_ref[...] = slot2.astype(jnp.int32)
    w0_ref[...] = jnp.broadcast_to(m1, (T, 16))
    w1_ref[...] = jnp.broadcast_to(m2, (T, 16))

    ends = off + cnt_pad
    ends_sq = jnp.dot(jnp.ones((E, 1), jnp.float32), ends,
                      preferred_element_type=jnp.float32)
    ends_col = jnp.sum(jnp.where(er == ec, ends_sq, 0.0),
                       axis=-1, keepdims=True)
    starts = (lax.broadcasted_iota(jnp.int32, (1, NT), 1) * BT)
    passed = (starts.astype(jnp.float32) >= ends_col)
    te_ref[...] = jnp.sum(passed.astype(jnp.int32), axis=0, keepdims=True)


def _dispatch_body(x_hbm, s0_hbm, s1_hbm, xs_hbm,
                   idx0_v, idx1_v, rows_v, sem0, sem1):
    w = lax.axis_index("s") * 2 + lax.axis_index("c")
    base = w * TPW
    pltpu.sync_copy(s0_hbm.at[pl.ds(base, TPW)], idx0_v)
    pltpu.sync_copy(s1_hbm.at[pl.ds(base, TPW)], idx1_v)
    pltpu.sync_copy(x_hbm.at[pl.ds(base, TPW)], rows_v)
    cp0 = pltpu.async_copy(rows_v, xs_hbm.at[idx0_v], sem0)
    cp1 = pltpu.async_copy(rows_v, xs_hbm.at[idx1_v], sem1)
    cp0.wait()
    cp1.wait()


_dispatch = functools.partial(
    pl.kernel,
    out_type=jax.ShapeDtypeStruct((PAD, D), jnp.float32),
    mesh=plsc.VectorSubcoreMesh(core_axis_name="c", subcore_axis_name="s"),
    scratch_types=[
        pltpu.VMEM((TPW,), jnp.int32),
        pltpu.VMEM((TPW,), jnp.int32),
        pltpu.VMEM((TPW, D), jnp.float32),
        pltpu.SemaphoreType.DMA,
        pltpu.SemaphoreType.DMA,
    ],
)(_dispatch_body)


def _gemm_body(te_ref, xs_ref, w1_ref, w2_ref, ys_ref, acc_ref):
    kh = pl.program_id(0)
    i = pl.program_id(1)

    @pl.when(te_ref[i] < E)
    def _():
        h = jnp.dot(xs_ref[...], w1_ref[0], preferred_element_type=jnp.float32)
        h = h * jax.nn.sigmoid(h)
        contrib = jnp.dot(h, w2_ref[0], preferred_element_type=jnp.float32)
        sl = pl.ds(i * BT, BT)

        @pl.when(kh == 0)
        def _():
            acc_ref[sl, :] = contrib

        @pl.when(kh > 0)
        def _():
            acc_ref[sl, :] += contrib

        @pl.when(kh == KH - 1)
        def _():
            ys_ref[...] = acc_ref[sl, :]


def _gemm(te, xs, w1, w2):
    def clamp(v):
        return jnp.minimum(v, E - 1)

    return pl.pallas_call(
        _gemm_body,
        grid_spec=pltpu.PrefetchScalarGridSpec(
            num_scalar_prefetch=1,
            grid=(KH, NT),
            in_specs=[
                pl.BlockSpec((BT, D), lambda kh, i, te: (i, 0)),
                pl.BlockSpec((1, D, HC), lambda kh, i, te: (clamp(te[i]), 0, kh)),
                pl.BlockSpec((1, HC, D), lambda kh, i, te: (clamp(te[i]), kh, 0)),
            ],
            out_specs=pl.BlockSpec((BT, D), lambda kh, i, te: (i, 0)),
            scratch_shapes=[pltpu.VMEM((PAD, D), jnp.float32)],
        ),
        out_shape=jax.ShapeDtypeStruct((PAD, D), jnp.float32),
    )(te, xs, w1, w2)


_CCH = 32


def _combine_body(ys_hbm, s0_hbm, s1_hbm, w0_hbm, w1_hbm, out_hbm,
                  idx0_v, idx1_v, wt0_v, wt1_v, buf0_v, buf1_v, sem0, sem1):
    w = lax.axis_index("s") * 2 + lax.axis_index("c")
    for ch in range(TPW // _CCH):
        base = w * TPW + ch * _CCH
        pltpu.sync_copy(s0_hbm.at[pl.ds(base, _CCH)], idx0_v)
        pltpu.sync_copy(s1_hbm.at[pl.ds(base, _CCH)], idx1_v)
        pltpu.sync_copy(w0_hbm.at[pl.ds(base, _CCH)], wt0_v)
        pltpu.sync_copy(w1_hbm.at[pl.ds(base, _CCH)], wt1_v)
        cp0 = pltpu.async_copy(ys_hbm.at[idx0_v], buf0_v, sem0)
        cp1 = pltpu.async_copy(ys_hbm.at[idx1_v], buf1_v, sem1)
        cp0.wait()
        cp1.wait()

        def row(i, _):
            w0 = wt0_v[i, :]
            w1v = wt1_v[i, :]
            for c in range(D // 16):
                s = pl.ds(c * 16, 16)
                buf0_v[i, s] = w0 * buf0_v[i, s] + w1v * buf1_v[i, s]
            return 0

        lax.fori_loop(0, _CCH, row, 0)
        pltpu.sync_copy(buf0_v, out_hbm.at[pl.ds(base, _CCH)])


_combine = functools.partial(
    pl.kernel,
    out_type=jax.ShapeDtypeStruct((T, D), jnp.float32),
    mesh=plsc.VectorSubcoreMesh(core_axis_name="c", subcore_axis_name="s"),
    scratch_types=[
        pltpu.VMEM((_CCH,), jnp.int32),
        pltpu.VMEM((_CCH,), jnp.int32),
        pltpu.VMEM((_CCH, 16), jnp.float32),
        pltpu.VMEM((_CCH, 16), jnp.float32),
        pltpu.VMEM((_CCH, D), jnp.float32),
        pltpu.VMEM((_CCH, D), jnp.float32),
        pltpu.SemaphoreType.DMA,
        pltpu.SemaphoreType.DMA,
    ],
)(_combine_body)


@jax.jit
def kernel(x, router_w, w1, w2):
    s0, s1, wt0, wt1, te = pl.pallas_call(
        _router_body,
        out_shape=(
            jax.ShapeDtypeStruct((T, 1), jnp.int32),
            jax.ShapeDtypeStruct((T, 1), jnp.int32),
            jax.ShapeDtypeStruct((T, 16), jnp.float32),
            jax.ShapeDtypeStruct((T, 16), jnp.float32),
            jax.ShapeDtypeStruct((1, NT), jnp.int32),
        ),
    )(x, router_w)

    s0f = s0.reshape(T)
    s1f = s1.reshape(T)
    xs = _dispatch(x, s0f, s1f)
    ys = _gemm(te.reshape(NT), xs, w1, w2)
    out = _combine(ys, s0f, s1f, wt0, wt1)
    return out


# speedup vs baseline: 1.3226x; 1.1011x over previous
import functools

import jax
import jax.numpy as jnp
from jax import lax
from jax.experimental import pallas as pl
from jax.experimental.pallas import tpu as pltpu
from jax.experimental.pallas import tpu_sc as plsc

T = 2048
D = 1024
E = 8
H = 4096
K = 2
KH = 4
HC = H // KH
BT = 256
NT = 24
PAD = NT * BT

NSUB = 32
TPW = T // NSUB


def _router_body(x_ref, rw_ref, s0_ref, s1_ref, w0_ref, w1_ref, te_ref):
    x = x_ref[...]
    logits = jnp.dot(x, rw_ref[...], preferred_element_type=jnp.float32)
    m = jnp.max(logits, axis=-1, keepdims=True)
    ex = jnp.exp(logits - m)
    scores = ex / jnp.sum(ex, axis=-1, keepdims=True)

    eids = lax.broadcasted_iota(jnp.int32, (T, E), 1)
    m1 = jnp.max(scores, axis=-1, keepdims=True)
    i1 = jnp.argmax(scores, axis=-1)[:, None]
    masked = jnp.where(eids == i1, -jnp.inf, scores)
    m2 = jnp.max(masked, axis=-1, keepdims=True)
    i2 = jnp.argmax(masked, axis=-1)[:, None]

    sel1 = (eids == i1)
    sel2 = (eids == i2)
    mask = (sel1 | sel2).astype(jnp.float32)

    r = lax.broadcasted_iota(jnp.int32, (T, T), 0)
    c = lax.broadcasted_iota(jnp.int32, (T, T), 1)
    ltri = (c < r).astype(jnp.bfloat16)
    excl = jnp.dot(ltri, mask.astype(jnp.bfloat16),
                   preferred_element_type=jnp.float32)

    counts = jnp.sum(mask, axis=0, keepdims=True)
    cnt_pad = (jnp.ceil(counts / BT) * BT).astype(jnp.float32)

    er = lax.broadcasted_iota(jnp.int32, (E, E), 0)
    ec = lax.broadcasted_iota(jnp.int32, (E, E), 1)
    off = jnp.dot(cnt_pad, (er < ec).astype(jnp.float32),
                  preferred_element_type=jnp.float32)

    slot_base = off + excl
    slot1 = jnp.sum(jnp.where(sel1, slot_base, 0.0), axis=-1, keepdims=True)
    slot2 = jnp.sum(jnp.where(sel2, slot_base, 0.0), axis=-1, keepdims=True)
    s0_ref[...] = slot1.astype(jnp.int32)
    s1_ref[...] = slot2.astype(jnp.int32)
    w0_ref[...] = jnp.broadcast_to(m1, (T, 16))
    w1_ref[...] = jnp.broadcast_to(m2, (T, 16))

    ends = off + cnt_pad
    ends_sq = jnp.dot(jnp.ones((E, 1), jnp.float32), ends,
                      preferred_element_type=jnp.float32)
    ends_col = jnp.sum(jnp.where(er == ec, ends_sq, 0.0),
                       axis=-1, keepdims=True)
    starts = (lax.broadcasted_iota(jnp.int32, (1, NT), 1) * BT)
    passed = (starts.astype(jnp.float32) >= ends_col)
    te_ref[...] = jnp.sum(passed.astype(jnp.int32), axis=0, keepdims=True)


def _dispatch_body(x_hbm, s0_hbm, s1_hbm, xs_hbm,
                   idx0_v, idx1_v, rows_v, sem0, sem1):
    w = lax.axis_index("s") * 2 + lax.axis_index("c")
    base = w * TPW
    pltpu.sync_copy(s0_hbm.at[pl.ds(base, TPW)], idx0_v)
    pltpu.sync_copy(s1_hbm.at[pl.ds(base, TPW)], idx1_v)
    pltpu.sync_copy(x_hbm.at[pl.ds(base, TPW)], rows_v)
    cp0 = pltpu.async_copy(rows_v, xs_hbm.at[idx0_v], sem0)
    cp1 = pltpu.async_copy(rows_v, xs_hbm.at[idx1_v], sem1)
    cp0.wait()
    cp1.wait()


_dispatch = functools.partial(
    pl.kernel,
    out_type=jax.ShapeDtypeStruct((PAD, D), jnp.float32),
    mesh=plsc.VectorSubcoreMesh(core_axis_name="c", subcore_axis_name="s"),
    scratch_types=[
        pltpu.VMEM((TPW,), jnp.int32),
        pltpu.VMEM((TPW,), jnp.int32),
        pltpu.VMEM((TPW, D), jnp.float32),
        pltpu.SemaphoreType.DMA,
        pltpu.SemaphoreType.DMA,
    ],
)(_dispatch_body)


def _gemm_body(te_ref, xs_ref, w1_ref, w2_ref, ys_ref, acc_ref):
    kh = pl.program_id(0)
    i = pl.program_id(1)

    @pl.when(te_ref[i] < E)
    def _():
        h = jnp.dot(xs_ref[...], w1_ref[0], preferred_element_type=jnp.float32)
        h = h * jax.nn.sigmoid(h)
        contrib = jnp.dot(h, w2_ref[0], preferred_element_type=jnp.float32)
        sl = pl.ds(i * BT, BT)

        @pl.when(kh == 0)
        def _():
            acc_ref[sl, :] = contrib

        @pl.when(kh > 0)
        def _():
            acc_ref[sl, :] += contrib

        @pl.when(kh == KH - 1)
        def _():
            ys_ref[...] = acc_ref[sl, :]


def _gemm(te, xs, w1, w2):
    def clamp(v):
        return jnp.minimum(v, E - 1)

    return pl.pallas_call(
        _gemm_body,
        grid_spec=pltpu.PrefetchScalarGridSpec(
            num_scalar_prefetch=1,
            grid=(KH, NT),
            in_specs=[
                pl.BlockSpec((BT, D), lambda kh, i, te: (i, 0)),
                pl.BlockSpec((1, D, HC), lambda kh, i, te: (clamp(te[i]), 0, kh)),
                pl.BlockSpec((1, HC, D), lambda kh, i, te: (clamp(te[i]), kh, 0)),
            ],
            out_specs=pl.BlockSpec((BT, D), lambda kh, i, te: (i, 0)),
            scratch_shapes=[pltpu.VMEM((PAD, D), jnp.float32)],
        ),
        out_shape=jax.ShapeDtypeStruct((PAD, D), jnp.float32),
    )(te, xs, w1, w2)


_CCH = 32


def _combine_body(ys_hbm, s0_hbm, s1_hbm, w0_hbm, w1_hbm, out_hbm,
                  idx0_v, idx1_v, wt0_v, wt1_v, buf0_v, buf1_v, sem0, sem1):
    w = lax.axis_index("s") * 2 + lax.axis_index("c")
    for ch in range(TPW // _CCH):
        base = w * TPW + ch * _CCH
        pltpu.sync_copy(s0_hbm.at[pl.ds(base, _CCH)], idx0_v)
        pltpu.sync_copy(s1_hbm.at[pl.ds(base, _CCH)], idx1_v)
        pltpu.sync_copy(w0_hbm.at[pl.ds(base, _CCH)], wt0_v)
        pltpu.sync_copy(w1_hbm.at[pl.ds(base, _CCH)], wt1_v)
        cp0 = pltpu.async_copy(ys_hbm.at[idx0_v], buf0_v, sem0)
        cp1 = pltpu.async_copy(ys_hbm.at[idx1_v], buf1_v, sem1)
        cp0.wait()
        cp1.wait()

        def row(i, _):
            w0 = wt0_v[i, :]
            w1v = wt1_v[i, :]
            for c in range(D // 16):
                s = pl.ds(c * 16, 16)
                buf0_v[i, s] = w0 * buf0_v[i, s] + w1v * buf1_v[i, s]
            return 0

        lax.fori_loop(0, _CCH, row, 0)
        pltpu.sync_copy(buf0_v, out_hbm.at[pl.ds(base, _CCH)])


_combine = functools.partial(
    pl.kernel,
    out_type=jax.ShapeDtypeStruct((T, D), jnp.float32),
    mesh=plsc.VectorSubcoreMesh(core_axis_name="c", subcore_axis_name="s"),
    scratch_types=[
        pltpu.VMEM((_CCH,), jnp.int32),
        pltpu.VMEM((_CCH,), jnp.int32),
        pltpu.VMEM((_CCH, 16), jnp.float32),
        pltpu.VMEM((_CCH, 16), jnp.float32),
        pltpu.VMEM((_CCH, D), jnp.float32),
        pltpu.VMEM((_CCH, D), jnp.float32),
        pltpu.SemaphoreType.DMA,
        pltpu.SemaphoreType.DMA,
    ],
)(_combine_body)


@jax.jit
def kernel(x, router_w, w1, w2):
    s0, s1, wt0, wt1, te = pl.pallas_call(
        _router_body,
        out_shape=(
            jax.ShapeDtypeStruct((T, 1), jnp.int32),
            jax.ShapeDtypeStruct((T, 1), jnp.int32),
            jax.ShapeDtypeStruct((T, 16), jnp.float32),
            jax.ShapeDtypeStruct((T, 16), jnp.float32),
            jax.ShapeDtypeStruct((1, NT), jnp.int32),
        ),
    )(x, router_w)

    s0f = s0.reshape(T)
    s1f = s1.reshape(T)
    xs = _dispatch(x, s0f, s1f)
    ys = _gemm(te.reshape(NT), xs, w1, w2)
    out = _combine(ys, s0f, s1f, wt0, wt1)
    return out


# speedup vs baseline: 1.3996x; 1.0582x over previous
import functools

import jax
import jax.numpy as jnp
from jax import lax
from jax.experimental import pallas as pl
from jax.experimental.pallas import tpu as pltpu
from jax.experimental.pallas import tpu_sc as plsc

T = 2048
D = 1024
E = 8
H = 4096
K = 2
KH = 4
HC = H // KH
BT = 256
NT = 24
PAD = NT * BT

NSUB = 32
TPW = T // NSUB


def _router_body(x_ref, rw_ref, s0_ref, s1_ref, w0_ref, w1_ref, te_ref):
    x = x_ref[...]
    logits = jnp.dot(x, rw_ref[...], preferred_element_type=jnp.float32)
    m = jnp.max(logits, axis=-1, keepdims=True)
    ex = jnp.exp(logits - m)
    scores = ex / jnp.sum(ex, axis=-1, keepdims=True)

    eids = lax.broadcasted_iota(jnp.int32, (T, E), 1)
    m1 = jnp.max(scores, axis=-1, keepdims=True)
    i1 = jnp.argmax(scores, axis=-1)[:, None]
    masked = jnp.where(eids == i1, -jnp.inf, scores)
    m2 = jnp.max(masked, axis=-1, keepdims=True)
    i2 = jnp.argmax(masked, axis=-1)[:, None]

    sel1 = (eids == i1)
    sel2 = (eids == i2)
    mask = (sel1 | sel2).astype(jnp.float32)

    r = lax.broadcasted_iota(jnp.int32, (T, T), 0)
    c = lax.broadcasted_iota(jnp.int32, (T, T), 1)
    ltri = (c < r).astype(jnp.bfloat16)
    excl = jnp.dot(ltri, mask.astype(jnp.bfloat16),
                   preferred_element_type=jnp.float32)

    counts = jnp.sum(mask, axis=0, keepdims=True)
    cnt_pad = (jnp.ceil(counts / BT) * BT).astype(jnp.float32)

    er = lax.broadcasted_iota(jnp.int32, (E, E), 0)
    ec = lax.broadcasted_iota(jnp.int32, (E, E), 1)
    off = jnp.dot(cnt_pad, (er < ec).astype(jnp.float32),
                  preferred_element_type=jnp.float32)

    slot_base = off + excl
    slot1 = jnp.sum(jnp.where(sel1, slot_base, 0.0), axis=-1, keepdims=True)
    slot2 = jnp.sum(jnp.where(sel2, slot_base, 0.0), axis=-1, keepdims=True)
    s0_ref[...] = slot1.astype(jnp.int32)
    s1_ref[...] = slot2.astype(jnp.int32)
    w0_ref[...] = jnp.broadcast_to(m1, (T, 16))
    w1_ref[...] = jnp.broadcast_to(m2, (T, 16))

    ends = off + cnt_pad
    ends_sq = jnp.dot(jnp.ones((E, 1), jnp.float32), ends,
                      preferred_element_type=jnp.float32)
    ends_col = jnp.sum(jnp.where(er == ec, ends_sq, 0.0),
                       axis=-1, keepdims=True)
    starts = (lax.broadcasted_iota(jnp.int32, (1, NT), 1) * BT)
    passed = (starts.astype(jnp.float32) >= ends_col)
    te_ref[...] = jnp.sum(passed.astype(jnp.int32), axis=0, keepdims=True)


def _dispatch_body(x_hbm, s0_hbm, s1_hbm, xs_hbm,
                   idx0_v, idx1_v, rows_v, sem0, sem1):
    w = lax.axis_index("s") * 2 + lax.axis_index("c")
    base = w * TPW
    pltpu.sync_copy(s0_hbm.at[pl.ds(base, TPW)], idx0_v)
    pltpu.sync_copy(s1_hbm.at[pl.ds(base, TPW)], idx1_v)
    pltpu.sync_copy(x_hbm.at[pl.ds(base, TPW)], rows_v)
    cp0 = pltpu.async_copy(rows_v, xs_hbm.at[idx0_v], sem0)
    cp1 = pltpu.async_copy(rows_v, xs_hbm.at[idx1_v], sem1)
    cp0.wait()
    cp1.wait()


_dispatch = functools.partial(
    pl.kernel,
    out_type=jax.ShapeDtypeStruct((PAD, D), jnp.float32),
    mesh=plsc.VectorSubcoreMesh(core_axis_name="c", subcore_axis_name="s"),
    scratch_types=[
        pltpu.VMEM((TPW,), jnp.int32),
        pltpu.VMEM((TPW,), jnp.int32),
        pltpu.VMEM((TPW, D), jnp.float32),
        pltpu.SemaphoreType.DMA,
        pltpu.SemaphoreType.DMA,
    ],
)(_dispatch_body)


def _gemm_body(te_ref, xs_ref, w1_ref, w2_ref, ys_ref, acc_ref):
    kh = pl.program_id(0)
    i = pl.program_id(1)

    @pl.when(te_ref[i] < E)
    def _():
        h = jnp.dot(xs_ref[...], w1_ref[0], preferred_element_type=jnp.float32)
        h = h * jax.nn.sigmoid(h)
        contrib = jnp.dot(h, w2_ref[0], preferred_element_type=jnp.float32)
        sl = pl.ds(i * BT, BT)

        @pl.when(kh == 0)
        def _():
            acc_ref[sl, :] = contrib

        @pl.when(kh > 0)
        def _():
            acc_ref[sl, :] += contrib

        @pl.when(kh == KH - 1)
        def _():
            ys_ref[...] = acc_ref[sl, :]


def _gemm(te, xs, w1, w2):
    def clamp(v):
        return jnp.minimum(v, E - 1)

    return pl.pallas_call(
        _gemm_body,
        grid_spec=pltpu.PrefetchScalarGridSpec(
            num_scalar_prefetch=1,
            grid=(KH, NT),
            in_specs=[
                pl.BlockSpec((BT, D), lambda kh, i, te: (i, 0)),
                pl.BlockSpec((1, D, HC), lambda kh, i, te: (clamp(te[i]), 0, kh)),
                pl.BlockSpec((1, HC, D), lambda kh, i, te: (clamp(te[i]), kh, 0)),
            ],
            out_specs=pl.BlockSpec(
                (BT, D),
                lambda kh, i, te: (jnp.where(kh == KH - 1, i, 0), 0)),
            scratch_shapes=[pltpu.VMEM((PAD, D), jnp.float32)],
        ),
        out_shape=jax.ShapeDtypeStruct((PAD, D), jnp.float32),
    )(te, xs, w1, w2)


_CCH = 32


def _combine_body(ys_hbm, s0_hbm, s1_hbm, w0_hbm, w1_hbm, out_hbm,
                  idx0_v, idx1_v, wt0_v, wt1_v, buf0_v, buf1_v, sem0, sem1):
    w = lax.axis_index("s") * 2 + lax.axis_index("c")
    for ch in range(TPW // _CCH):
        base = w * TPW + ch * _CCH
        pltpu.sync_copy(s0_hbm.at[pl.ds(base, _CCH)], idx0_v)
        pltpu.sync_copy(s1_hbm.at[pl.ds(base, _CCH)], idx1_v)
        pltpu.sync_copy(w0_hbm.at[pl.ds(base, _CCH)], wt0_v)
        pltpu.sync_copy(w1_hbm.at[pl.ds(base, _CCH)], wt1_v)
        cp0 = pltpu.async_copy(ys_hbm.at[idx0_v], buf0_v, sem0)
        cp1 = pltpu.async_copy(ys_hbm.at[idx1_v], buf1_v, sem1)
        cp0.wait()
        cp1.wait()

        def row(i, _):
            w0 = wt0_v[i, :]
            w1v = wt1_v[i, :]
            for c in range(D // 16):
                s = pl.ds(c * 16, 16)
                buf0_v[i, s] = w0 * buf0_v[i, s] + w1v * buf1_v[i, s]
            return 0

        lax.fori_loop(0, _CCH, row, 0)
        pltpu.sync_copy(buf0_v, out_hbm.at[pl.ds(base, _CCH)])


_combine = functools.partial(
    pl.kernel,
    out_type=jax.ShapeDtypeStruct((T, D), jnp.float32),
    mesh=plsc.VectorSubcoreMesh(core_axis_name="c", subcore_axis_name="s"),
    scratch_types=[
        pltpu.VMEM((_CCH,), jnp.int32),
        pltpu.VMEM((_CCH,), jnp.int32),
        pltpu.VMEM((_CCH, 16), jnp.float32),
        pltpu.VMEM((_CCH, 16), jnp.float32),
        pltpu.VMEM((_CCH, D), jnp.float32),
        pltpu.VMEM((_CCH, D), jnp.float32),
        pltpu.SemaphoreType.DMA,
        pltpu.SemaphoreType.DMA,
    ],
)(_combine_body)


@jax.jit
def kernel(x, router_w, w1, w2):
    s0, s1, wt0, wt1, te = pl.pallas_call(
        _router_body,
        out_shape=(
            jax.ShapeDtypeStruct((T, 1), jnp.int32),
            jax.ShapeDtypeStruct((T, 1), jnp.int32),
            jax.ShapeDtypeStruct((T, 16), jnp.float32),
            jax.ShapeDtypeStruct((T, 16), jnp.float32),
            jax.ShapeDtypeStruct((1, NT), jnp.int32),
        ),
    )(x, router_w)

    s0f = s0.reshape(T)
    s1f = s1.reshape(T)
    xs = _dispatch(x, s0f, s1f)
    ys = _gemm(te.reshape(NT), xs, w1, w2)
    out = _combine(ys, s0f, s1f, wt0, wt1)
    return out


# speedup vs baseline: 1.4632x; 1.0454x over previous
import functools

import jax
import jax.numpy as jnp
from jax import lax
from jax.experimental import pallas as pl
from jax.experimental.pallas import tpu as pltpu
from jax.experimental.pallas import tpu_sc as plsc

T = 2048
D = 1024
E = 8
H = 4096
K = 2
KH = 4
HC = H // KH
BT = 256
NT = 24
PAD = NT * BT

NSUB = 32
TPW = T // NSUB


def _router_body(x_ref, rw_ref, s0_ref, s1_ref, w0_ref, w1_ref, te_ref):
    x = x_ref[...]
    logits = jnp.dot(x, rw_ref[...], preferred_element_type=jnp.float32)
    m = jnp.max(logits, axis=-1, keepdims=True)
    ex = jnp.exp(logits - m)
    scores = ex / jnp.sum(ex, axis=-1, keepdims=True)

    eids = lax.broadcasted_iota(jnp.int32, (T, E), 1)
    m1 = jnp.max(scores, axis=-1, keepdims=True)
    i1 = jnp.argmax(scores, axis=-1)[:, None]
    masked = jnp.where(eids == i1, -jnp.inf, scores)
    m2 = jnp.max(masked, axis=-1, keepdims=True)
    i2 = jnp.argmax(masked, axis=-1)[:, None]

    sel1 = (eids == i1)
    sel2 = (eids == i2)
    mask = (sel1 | sel2).astype(jnp.float32)

    r = lax.broadcasted_iota(jnp.int32, (T, T), 0)
    c = lax.broadcasted_iota(jnp.int32, (T, T), 1)
    ltri = (c < r).astype(jnp.bfloat16)
    excl = jnp.dot(ltri, mask.astype(jnp.bfloat16),
                   preferred_element_type=jnp.float32)

    counts = jnp.sum(mask, axis=0, keepdims=True)
    cnt_pad = (jnp.ceil(counts / BT) * BT).astype(jnp.float32)

    er = lax.broadcasted_iota(jnp.int32, (E, E), 0)
    ec = lax.broadcasted_iota(jnp.int32, (E, E), 1)
    off = jnp.dot(cnt_pad, (er < ec).astype(jnp.float32),
                  preferred_element_type=jnp.float32)

    slot_base = off + excl
    slot1 = jnp.sum(jnp.where(sel1, slot_base, 0.0), axis=-1, keepdims=True)
    slot2 = jnp.sum(jnp.where(sel2, slot_base, 0.0), axis=-1, keepdims=True)
    s0_ref[...] = slot1.astype(jnp.int32)
    s1_ref[...] = slot2.astype(jnp.int32)
    w0_ref[...] = jnp.broadcast_to(m1, (T, 16))
    w1_ref[...] = jnp.broadcast_to(m2, (T, 16))

    ends = off + cnt_pad
    ends_sq = jnp.dot(jnp.ones((E, 1), jnp.float32), ends,
                      preferred_element_type=jnp.float32)
    ends_col = jnp.sum(jnp.where(er == ec, ends_sq, 0.0),
                       axis=-1, keepdims=True)
    starts = (lax.broadcasted_iota(jnp.int32, (1, NT), 1) * BT)
    passed = (starts.astype(jnp.float32) >= ends_col)
    te_ref[...] = jnp.sum(passed.astype(jnp.int32), axis=0, keepdims=True)


def _dispatch_body(x_hbm, s0_hbm, s1_hbm, xs_hbm,
                   idx0_v, idx1_v, rows_v, sem0, sem1):
    w = lax.axis_index("s") * 2 + lax.axis_index("c")
    base = w * TPW
    pltpu.sync_copy(s0_hbm.at[pl.ds(base, TPW)], idx0_v)
    pltpu.sync_copy(s1_hbm.at[pl.ds(base, TPW)], idx1_v)
    pltpu.sync_copy(x_hbm.at[pl.ds(base, TPW)], rows_v)
    cp0 = pltpu.async_copy(rows_v, xs_hbm.at[idx0_v], sem0)
    cp1 = pltpu.async_copy(rows_v, xs_hbm.at[idx1_v], sem1)
    cp0.wait()
    cp1.wait()


_dispatch = functools.partial(
    pl.kernel,
    out_type=jax.ShapeDtypeStruct((PAD, D), jnp.float32),
    mesh=plsc.VectorSubcoreMesh(core_axis_name="c", subcore_axis_name="s"),
    scratch_types=[
        pltpu.VMEM((TPW,), jnp.int32),
        pltpu.VMEM((TPW,), jnp.int32),
        pltpu.VMEM((TPW, D), jnp.float32),
        pltpu.SemaphoreType.DMA,
        pltpu.SemaphoreType.DMA,
    ],
)(_dispatch_body)


def _gemm_body(te_ref, xs_ref, w1_ref, w2_ref, ys_ref, acc_ref, xsb_ref):
    kh = pl.program_id(0)
    i = pl.program_id(1)

    @pl.when(te_ref[i] < E)
    def _():
        sl = pl.ds(i * BT, BT)

        @pl.when(kh == 0)
        def _():
            xsb_ref[sl, :] = xs_ref[...].astype(jnp.bfloat16)

        xb = xsb_ref[sl, :]
        h = jnp.dot(xb, w1_ref[0].astype(jnp.bfloat16),
                    preferred_element_type=jnp.float32)
        h = h * jax.nn.sigmoid(h)
        contrib = jnp.dot(h.astype(jnp.bfloat16),
                          w2_ref[0].astype(jnp.bfloat16),
                          preferred_element_type=jnp.float32)

        @pl.when(kh == 0)
        def _():
            acc_ref[sl, :] = contrib

        @pl.when(kh > 0)
        def _():
            acc_ref[sl, :] += contrib

        @pl.when(kh == KH - 1)
        def _():
            ys_ref[...] = acc_ref[sl, :]


def _gemm(te, xs, w1, w2):
    def clamp(v):
        return jnp.minimum(v, E - 1)

    return pl.pallas_call(
        _gemm_body,
        grid_spec=pltpu.PrefetchScalarGridSpec(
            num_scalar_prefetch=1,
            grid=(KH, NT),
            in_specs=[
                pl.BlockSpec(
                    (BT, D),
                    lambda kh, i, te: (jnp.where(kh == 0, i, 0), 0)),
                pl.BlockSpec((1, D, HC), lambda kh, i, te: (clamp(te[i]), 0, kh)),
                pl.BlockSpec((1, HC, D), lambda kh, i, te: (clamp(te[i]), kh, 0)),
            ],
            out_specs=pl.BlockSpec(
                (BT, D),
                lambda kh, i, te: (jnp.where(kh == KH - 1, i, 0), 0)),
            scratch_shapes=[pltpu.VMEM((PAD, D), jnp.float32),
                            pltpu.VMEM((PAD, D), jnp.bfloat16)],
        ),
        out_shape=jax.ShapeDtypeStruct((PAD, D), jnp.float32),
    )(te, xs, w1, w2)


_CCH = 32


def _combine_body(ys_hbm, s0_hbm, s1_hbm, w0_hbm, w1_hbm, out_hbm,
                  idx0_v, idx1_v, wt0_v, wt1_v, buf0_v, buf1_v, sem0, sem1):
    w = lax.axis_index("s") * 2 + lax.axis_index("c")
    for ch in range(TPW // _CCH):
        base = w * TPW + ch * _CCH
        pltpu.sync_copy(s0_hbm.at[pl.ds(base, _CCH)], idx0_v)
        pltpu.sync_copy(s1_hbm.at[pl.ds(base, _CCH)], idx1_v)
        pltpu.sync_copy(w0_hbm.at[pl.ds(base, _CCH)], wt0_v)
        pltpu.sync_copy(w1_hbm.at[pl.ds(base, _CCH)], wt1_v)
        cp0 = pltpu.async_copy(ys_hbm.at[idx0_v], buf0_v, sem0)
        cp1 = pltpu.async_copy(ys_hbm.at[idx1_v], buf1_v, sem1)
        cp0.wait()
        cp1.wait()

        def row(i, _):
            w0 = wt0_v[i, :]
            w1v = wt1_v[i, :]
            for c in range(D // 16):
                s = pl.ds(c * 16, 16)
                buf0_v[i, s] = w0 * buf0_v[i, s] + w1v * buf1_v[i, s]
            return 0

        lax.fori_loop(0, _CCH, row, 0)
        pltpu.sync_copy(buf0_v, out_hbm.at[pl.ds(base, _CCH)])


_combine = functools.partial(
    pl.kernel,
    out_type=jax.ShapeDtypeStruct((T, D), jnp.float32),
    mesh=plsc.VectorSubcoreMesh(core_axis_name="c", subcore_axis_name="s"),
    scratch_types=[
        pltpu.VMEM((_CCH,), jnp.int32),
        pltpu.VMEM((_CCH,), jnp.int32),
        pltpu.VMEM((_CCH, 16), jnp.float32),
        pltpu.VMEM((_CCH, 16), jnp.float32),
        pltpu.VMEM((_CCH, D), jnp.float32),
        pltpu.VMEM((_CCH, D), jnp.float32),
        pltpu.SemaphoreType.DMA,
        pltpu.SemaphoreType.DMA,
    ],
)(_combine_body)


@jax.jit
def kernel(x, router_w, w1, w2):
    s0, s1, wt0, wt1, te = pl.pallas_call(
        _router_body,
        out_shape=(
            jax.ShapeDtypeStruct((T, 1), jnp.int32),
            jax.ShapeDtypeStruct((T, 1), jnp.int32),
            jax.ShapeDtypeStruct((T, 16), jnp.float32),
            jax.ShapeDtypeStruct((T, 16), jnp.float32),
            jax.ShapeDtypeStruct((1, NT), jnp.int32),
        ),
    )(x, router_w)

    s0f = s0.reshape(T)
    s1f = s1.reshape(T)
    xs = _dispatch(x, s0f, s1f)
    ys = _gemm(te.reshape(NT), xs, w1, w2)
    out = _combine(ys, s0f, s1f, wt0, wt1)
    return out
